# Initial kernel scaffold; baseline (speedup 1.0000x reference)
import jax
import jax.numpy as jnp
from jax.experimental import pallas as pl


def kernel(x, x_lengths, emb):
    raise NotImplementedError("write your pallas kernel here")


# speedup vs baseline: 4.1417x; 4.1417x over previous
import functools
import jax
import jax.numpy as jnp
from jax import lax
from jax.experimental import pallas as pl
from jax.experimental.pallas import tpu as pltpu
from jax.experimental.pallas import tpu_sc as plsc

_N_VOCAB = 1000
_HIDDEN = 128
_B = 1024
_T = 200

_NW = 32
_N = _B * _T
_NPW = _N // _NW
_C = 128
_NCHUNK = _NPW // _C


def _gather_sc(x_flat, emb):
    mesh = plsc.VectorSubcoreMesh(core_axis_name="c", subcore_axis_name="s")

    @functools.partial(
        pl.kernel,
        mesh=mesh,
        out_type=jax.ShapeDtypeStruct((_N, _HIDDEN), jnp.float32),
        scratch_types=[
            pltpu.VMEM((_C,), jnp.int32),
            pltpu.VMEM((_C, _HIDDEN), jnp.float32),
            pltpu.SemaphoreType.DMA,
        ],
    )
    def k(x_hbm, emb_hbm, out_hbm, idx_v, rows_v, sem):
        wid = lax.axis_index("s") * 2 + lax.axis_index("c")
        base = wid * _NPW

        def body(j, carry):
            off = base + j * _C
            pltpu.sync_copy(x_hbm.at[pl.ds(off, _C)], idx_v)
            pltpu.async_copy(emb_hbm.at[idx_v], rows_v, sem).wait()
            pltpu.sync_copy(rows_v, out_hbm.at[pl.ds(off, _C)])
            return carry

        lax.fori_loop(0, _NCHUNK, body, 0)

    return k(x_flat, emb)


def _mask_tc(x_lengths):
    def mask_kernel(len_ref, out_ref):
        t_idx = lax.broadcasted_iota(jnp.int32, (_B, 1, _T), 2)
        lens = len_ref[...].reshape(_B, 1, 1)
        out_ref[...] = (t_idx < lens).astype(jnp.float32)

    return pl.pallas_call(
        mask_kernel,
        out_shape=jax.ShapeDtypeStruct((_B, 1, _T), jnp.float32),
    )(x_lengths.reshape(_B, 1))


def kernel(x, x_lengths, emb):
    x_flat = x.reshape(_N).astype(jnp.int32)
    x_emb = _gather_sc(x_flat, emb).reshape(_B, _T, _HIDDEN)
    x_mask = _mask_tc(x_lengths)
    return (x_mask, x_emb)


# speedup vs baseline: 4.3453x; 1.0492x over previous
import functools
import jax
import jax.numpy as jnp
from jax import lax
from jax.experimental import pallas as pl
from jax.experimental.pallas import tpu as pltpu
from jax.experimental.pallas import tpu_sc as plsc

_N_VOCAB = 1000
_HIDDEN = 128
_B = 1024
_T = 200

_NW = 32
_N = _B * _T
_NPW = _N // _NW
_C = 128
_NCHUNK = _NPW // _C
_NBUF = 5
_AHEAD = 3
_NG = _NCHUNK // _NBUF


def _gather_sc(x2, emb):
    mesh = plsc.VectorSubcoreMesh(core_axis_name="c", subcore_axis_name="s")

    @functools.partial(
        pl.kernel,
        mesh=mesh,
        out_type=jax.ShapeDtypeStruct((_N, _HIDDEN), jnp.float32),
        scratch_types=(
            [pltpu.VMEM((_NCHUNK, _C), jnp.int32)]
            + [pltpu.VMEM((_C, _HIDDEN), jnp.float32)] * _NBUF
            + [pltpu.SemaphoreType.DMA] * (2 * _NBUF)
        ),
    )
    def k(x_hbm, emb_hbm, out_hbm, idx_v, *bufs_sems):
        rows = bufs_sems[:_NBUF]
        gsem = bufs_sems[_NBUF:2 * _NBUF]
        osem = bufs_sems[2 * _NBUF:]

        wid = lax.axis_index("s") * 2 + lax.axis_index("c")
        base = wid * _NCHUNK

        pltpu.sync_copy(x_hbm.at[wid], idx_v)

        def fire_gather(j, b):
            pltpu.make_async_copy(emb_hbm.at[idx_v.at[j]], rows[b],
                                  gsem[b]).start()

        for b in range(_AHEAD):
            fire_gather(b, b)

        def outer(g, carry):
            for b in range(_NBUF):
                j = g * _NBUF + b
                jn = j + _AHEAD
                bn = (b + _AHEAD) % _NBUF

                @pl.when(jnp.logical_and(jn < _NCHUNK, j >= _NBUF - _AHEAD))
                def _():
                    pltpu.make_async_copy(
                        rows[bn], out_hbm.at[pl.ds((base + jn - _NBUF) * _C, _C)],
                        osem[bn]).wait()

                @pl.when(jn < _NCHUNK)
                def _():
                    fire_gather(jn, bn)

                pltpu.make_async_copy(emb_hbm.at[idx_v.at[j]], rows[b],
                                      gsem[b]).wait()
                pltpu.make_async_copy(
                    rows[b], out_hbm.at[pl.ds((base + j) * _C, _C)],
                    osem[b]).start()
            return carry

        lax.fori_loop(0, _NG, outer, 0)

        for b in range(_NBUF):
            j = _NCHUNK - _NBUF + b
            pltpu.make_async_copy(
                rows[(j) % _NBUF], out_hbm.at[pl.ds((base + j) * _C, _C)],
                osem[j % _NBUF]).wait()

    return k(x2, emb)


def _mask_tc(x_lengths):
    def mask_kernel(len_ref, out_ref):
        t_idx = lax.broadcasted_iota(jnp.int32, (_B, 1, _T), 2)
        lens = len_ref[...].reshape(_B, 1, 1)
        out_ref[...] = (t_idx < lens).astype(jnp.float32)

    return pl.pallas_call(
        mask_kernel,
        out_shape=jax.ShapeDtypeStruct((_B, 1, _T), jnp.float32),
    )(x_lengths.reshape(_B, 1))


def kernel(x, x_lengths, emb):
    x3 = x.reshape(_NW, _NCHUNK, _C).astype(jnp.int32)
    x_emb = _gather_sc(x3, emb).reshape(_B, _T, _HIDDEN)
    x_mask = _mask_tc(x_lengths)
    return (x_mask, x_emb)


# speedup vs baseline: 11.7346x; 2.7005x over previous
import functools
import jax
import jax.numpy as jnp
from jax import lax
from jax.experimental import pallas as pl
from jax.experimental.pallas import tpu as pltpu
from jax.experimental.pallas import tpu_sc as plsc

_N_VOCAB = 1000
_HIDDEN = 128
_B = 1024
_T = 200

_NW = 32
_N = _B * _T
_NPW = _N // _NW
_C = 128
_NCHUNK = _NPW // _C
_NBUF = 5
_AHEAD = 3
_NG = _NCHUNK // _NBUF


def _gather_sc(x2, emb):
    mesh = plsc.VectorSubcoreMesh(core_axis_name="c", subcore_axis_name="s")

    @functools.partial(
        pl.kernel,
        mesh=mesh,
        out_type=jax.ShapeDtypeStruct((_N, _HIDDEN), jnp.float32),
        scratch_types=(
            [pltpu.VMEM((_NCHUNK, _C), jnp.int32),
             pltpu.VMEM_SHARED((_N_VOCAB, _HIDDEN), jnp.float32)]
            + [pltpu.VMEM((_C, _HIDDEN), jnp.float32)] * _NBUF
            + [pltpu.SemaphoreType.DMA] * (2 * _NBUF)
        ),
    )
    def k(x_hbm, emb_hbm, out_hbm, idx_v, emb_sh, *bufs_sems):
        rows = bufs_sems[:_NBUF]
        gsem = bufs_sems[_NBUF:2 * _NBUF]
        osem = bufs_sems[2 * _NBUF:]

        sid = lax.axis_index("s")
        wid = sid * 2 + lax.axis_index("c")
        base = wid * _NCHUNK

        @pl.when(sid == 0)
        def _():
            pltpu.sync_copy(emb_hbm, emb_sh)

        pltpu.sync_copy(x_hbm.at[wid], idx_v)
        plsc.subcore_barrier()

        def fire_gather(j, b):
            pltpu.make_async_copy(emb_sh.at[idx_v.at[j]], rows[b],
                                  gsem[b]).start()

        for b in range(_AHEAD):
            fire_gather(b, b)

        def outer(g, carry):
            for b in range(_NBUF):
                j = g * _NBUF + b
                jn = j + _AHEAD
                bn = (b + _AHEAD) % _NBUF

                @pl.when(jnp.logical_and(jn < _NCHUNK, j >= _NBUF - _AHEAD))
                def _():
                    pltpu.make_async_copy(
                        rows[bn], out_hbm.at[pl.ds((base + jn - _NBUF) * _C, _C)],
                        osem[bn]).wait()

                @pl.when(jn < _NCHUNK)
                def _():
                    fire_gather(jn, bn)

                pltpu.make_async_copy(emb_sh.at[idx_v.at[j]], rows[b],
                                      gsem[b]).wait()
                pltpu.make_async_copy(
                    rows[b], out_hbm.at[pl.ds((base + j) * _C, _C)],
                    osem[b]).start()
            return carry

        lax.fori_loop(0, _NG, outer, 0)

        for b in range(_NBUF):
            j = _NCHUNK - _NBUF + b
            pltpu.make_async_copy(
                rows[(j) % _NBUF], out_hbm.at[pl.ds((base + j) * _C, _C)],
                osem[j % _NBUF]).wait()

    return k(x2, emb)


def _mask_tc(x_lengths):
    def mask_kernel(len_ref, out_ref):
        t_idx = lax.broadcasted_iota(jnp.int32, (_B, 1, _T), 2)
        lens = len_ref[...].reshape(_B, 1, 1)
        out_ref[...] = (t_idx < lens).astype(jnp.float32)

    return pl.pallas_call(
        mask_kernel,
        out_shape=jax.ShapeDtypeStruct((_B, 1, _T), jnp.float32),
    )(x_lengths.reshape(_B, 1))


def kernel(x, x_lengths, emb):
    x3 = x.reshape(_NW, _NCHUNK, _C).astype(jnp.int32)
    x_emb = _gather_sc(x3, emb).reshape(_B, _T, _HIDDEN)
    x_mask = _mask_tc(x_lengths)
    return (x_mask, x_emb)


# speedup vs baseline: 11.7631x; 1.0024x over previous
import functools
import jax
import jax.numpy as jnp
from jax import lax
from jax.experimental import pallas as pl
from jax.experimental.pallas import tpu as pltpu
from jax.experimental.pallas import tpu_sc as plsc

_N_VOCAB = 1000
_HIDDEN = 128
_B = 1024
_T = 200

_NW = 32
_N = _B * _T
_NPW = _N // _NW
_C = 128
_NCHUNK = _NPW // _C
_NBUF = 5
_AHEAD = 3
_NG = _NCHUNK // _NBUF


def _gather_sc(x2, emb):
    mesh = plsc.VectorSubcoreMesh(core_axis_name="c", subcore_axis_name="s")

    @functools.partial(
        pl.kernel,
        mesh=mesh,
        out_type=jax.ShapeDtypeStruct((_N, _HIDDEN), jnp.float32),
        scratch_types=(
            [pltpu.VMEM((_NCHUNK, _C), jnp.int32),
             pltpu.VMEM_SHARED((_N_VOCAB, _HIDDEN), jnp.float32)]
            + [pltpu.VMEM((_C, _HIDDEN), jnp.float32)] * _NBUF
            + [pltpu.SemaphoreType.DMA] * (2 * _NBUF)
        ),
    )
    def k(x_hbm, emb_hbm, out_hbm, idx_v, emb_sh, *bufs_sems):
        rows = bufs_sems[:_NBUF]
        gsem = bufs_sems[_NBUF:2 * _NBUF]
        osem = bufs_sems[2 * _NBUF:]

        sid = lax.axis_index("s")
        wid = sid * 2 + lax.axis_index("c")
        base = wid * _NCHUNK

        @pl.when(sid == 0)
        def _():
            pltpu.sync_copy(emb_hbm, emb_sh)

        pltpu.sync_copy(x_hbm.at[wid], idx_v)
        plsc.subcore_barrier()

        def fire_gather(j, b):
            pltpu.make_async_copy(emb_sh.at[idx_v.at[j]], rows[b],
                                  gsem[b]).start()

        for b in range(_AHEAD):
            fire_gather(b, b)

        def outer(g, carry):
            for b in range(_NBUF):
                j = g * _NBUF + b
                jn = j + _AHEAD
                bn = (b + _AHEAD) % _NBUF

                @pl.when(jnp.logical_and(jn < _NCHUNK, j >= _NBUF - _AHEAD))
                def _():
                    pltpu.make_async_copy(
                        rows[bn], out_hbm.at[pl.ds((base + jn - _NBUF) * _C, _C)],
                        osem[bn]).wait()

                @pl.when(jn < _NCHUNK)
                def _():
                    fire_gather(jn, bn)

                pltpu.make_async_copy(emb_sh.at[idx_v.at[j]], rows[b],
                                      gsem[b]).wait()
                pltpu.make_async_copy(
                    rows[b], out_hbm.at[pl.ds((base + j) * _C, _C)],
                    osem[b]).start()
            return carry

        lax.fori_loop(0, _NG, outer, 0)

        for b in range(_NBUF):
            j = _NCHUNK - _NBUF + b
            pltpu.make_async_copy(
                rows[(j) % _NBUF], out_hbm.at[pl.ds((base + j) * _C, _C)],
                osem[j % _NBUF]).wait()

    return k(x2, emb)


def _mask_tc(x_lengths):
    def mask_kernel(len_ref, out_ref):
        t_idx = lax.broadcasted_iota(jnp.int32, (_B, _T), 1)
        lens = len_ref[...].reshape(_B, 1)
        out_ref[...] = (t_idx < lens).astype(jnp.float32)

    m = pl.pallas_call(
        mask_kernel,
        out_shape=jax.ShapeDtypeStruct((_B, _T), jnp.float32),
    )(x_lengths.reshape(_B, 1))
    return m[:, None, :]


def kernel(x, x_lengths, emb):
    x3 = x.reshape(_NW, _NCHUNK, _C).astype(jnp.int32)
    x_emb = _gather_sc(x3, emb).reshape(_B, _T, _HIDDEN)
    x_mask = _mask_tc(x_lengths)
    return (x_mask, x_emb)


# speedup vs baseline: 11.8231x; 1.0051x over previous
import functools
import jax
import jax.numpy as jnp
from jax import lax
from jax.experimental import pallas as pl
from jax.experimental.pallas import tpu as pltpu
from jax.experimental.pallas import tpu_sc as plsc

_N_VOCAB = 1000
_HIDDEN = 128
_B = 1024
_T = 200

_NW = 32
_N = _B * _T
_RPW = _B // _NW
_NCH = 2 * _RPW
_C0, _C1 = 128, _T - 128
_NBUF = 8
_AHEAD = 4
_NGRP = _NCH // _NBUF


def _gather_sc(x, emb):
    mesh = plsc.VectorSubcoreMesh(core_axis_name="c", subcore_axis_name="s")
    sizes = [_C0 if k % 2 == 0 else _C1 for k in range(_NBUF)]
    offs = [0 if k % 2 == 0 else _C0 for k in range(_NBUF)]

    @functools.partial(
        pl.kernel,
        mesh=mesh,
        out_type=jax.ShapeDtypeStruct((_N, _HIDDEN), jnp.float32),
        scratch_types=(
            [pltpu.VMEM((_RPW, _T), jnp.int32),
             pltpu.VMEM_SHARED((_N_VOCAB, _HIDDEN), jnp.float32)]
            + [pltpu.VMEM((sizes[k], _HIDDEN), jnp.float32) for k in range(_NBUF)]
            + [pltpu.SemaphoreType.DMA] * (2 * _NBUF)
        ),
    )
    def k(x_hbm, emb_hbm, out_hbm, idx_v, emb_sh, *bufs_sems):
        rows = bufs_sems[:_NBUF]
        gsem = bufs_sems[_NBUF:2 * _NBUF]
        osem = bufs_sems[2 * _NBUF:]

        sid = lax.axis_index("s")
        wid = sid * 2 + lax.axis_index("c")
        rowbase = wid * _RPW
        outbase = rowbase * _T

        @pl.when(sid == 0)
        def _():
            pltpu.sync_copy(emb_hbm, emb_sh)

        pltpu.sync_copy(x_hbm.at[pl.ds(rowbase, _RPW)], idx_v)
        plsc.subcore_barrier()

        def idx_slice(m, k):
            return idx_v.at[m // 2, pl.ds(offs[k % _NBUF], sizes[k % _NBUF])]

        def out_slice(m, k):
            return out_hbm.at[
                pl.ds(outbase + (m // 2) * _T + offs[k % _NBUF],
                      sizes[k % _NBUF])]

        def fire_gather(m, k):
            pltpu.make_async_copy(emb_sh.at[idx_slice(m, k)],
                                  rows[k % _NBUF], gsem[k % _NBUF]).start()

        for m in range(_AHEAD):
            fire_gather(m, m)

        def outer(g, carry):
            for k in range(_NBUF):
                m = g * _NBUF + k
                mn = m + _AHEAD
                kn = (k + _AHEAD) % _NBUF

                @pl.when(jnp.logical_and(mn < _NCH, m >= _AHEAD))
                def _():
                    pltpu.make_async_copy(rows[kn], out_slice(m - _AHEAD, kn),
                                          osem[kn]).wait()

                @pl.when(mn < _NCH)
                def _():
                    fire_gather(mn, kn)

                pltpu.make_async_copy(emb_sh.at[idx_slice(m, k)],
                                      rows[k], gsem[k]).wait()
                pltpu.make_async_copy(rows[k], out_slice(m, k),
                                      osem[k]).start()
            return carry

        lax.fori_loop(0, _NGRP, outer, 0)

        for k in range(_NBUF):
            m = _NCH - _NBUF + k
            pltpu.make_async_copy(rows[k], out_slice(m, k), osem[k]).wait()

    return k(x, emb)


def _mask_tc(x_lengths):
    def mask_kernel(len_ref, out_ref):
        t_idx = lax.broadcasted_iota(jnp.int32, (_B, _T), 1)
        lens = len_ref[...].reshape(_B, 1)
        out_ref[...] = (t_idx < lens).astype(jnp.float32)

    m = pl.pallas_call(
        mask_kernel,
        out_shape=jax.ShapeDtypeStruct((_B, _T), jnp.float32),
    )(x_lengths.reshape(_B, 1))
    return m[:, None, :]


def kernel(x, x_lengths, emb):
    x_emb = _gather_sc(x.astype(jnp.int32), emb).reshape(_B, _T, _HIDDEN)
    x_mask = _mask_tc(x_lengths)
    return (x_mask, x_emb)
